# baseline (speedup 1.0000x reference)
import functools

import jax
import jax.numpy as jnp
from jax import lax
from jax.experimental import pallas as pl
from jax.experimental.pallas import tpu as pltpu
from jax.experimental.pallas import tpu_sc as plsc

H = 128
N_NODES = 50000
E = 500000

NUM_TILES = 16
NUM_CORES = 2
NCHUNK = 4
C = 12544
OUT_ROWS = NCHUNK * C
TRASH = 8
ROWS_PER_TILE = C // NUM_TILES
SEG = 4096
NSEG = 8
EDGES_PER_TILE = SEG * NSEG
E_PAD = EDGES_PER_TILE * NUM_TILES
RB = 128
RBS = 7
BIG = 1 << 27
CNTW = 32
ADT = jnp.bfloat16


def _seg_sum_body(with_count, refs):
    if with_count:
        (feat_hbm, src_hbm, dst_hbm, z800, zc800, ones_hbm,
         out_hbm, cnt_out_hbm,
         selsrc_v, seldst_v, srcseg_v, dstseg_v, rows0_v, rows1_v, ones_v,
         acc_sh, cnt_sh, sem0, sem1, sem_s0, sem_s1) = refs
    else:
        (feat_hbm, src_hbm, dst_hbm, z800,
         out_hbm,
         selsrc_v, seldst_v, srcseg_v, dstseg_v, rows0_v, rows1_v,
         acc_sh, sem0, sem1, sem_s0, sem_s1) = refs
        cnt_out_hbm = zc800 = ones_hbm = ones_v = cnt_sh = None

    core = lax.axis_index("c")
    sid = lax.axis_index("s")
    iota = lax.iota(jnp.int32, 16)
    padsrc_vals = iota * 97 + sid * 16
    paddst_vals = C + (iota & 7)

    if with_count:
        pltpu.sync_copy(ones_hbm, ones_v)
    ebase = sid * EDGES_PER_TILE

    def chunk_body(k, _):
        lo = (core * (NCHUNK // NUM_CORES) + k) * C
        plsc.subcore_barrier()
        pltpu.sync_copy(z800, acc_sh.at[pl.ds(sid * ROWS_PER_TILE, ROWS_PER_TILE)])
        if with_count:
            pltpu.sync_copy(zc800, cnt_sh.at[pl.ds(sid * ROWS_PER_TILE, ROWS_PER_TILE)])
        plsc.subcore_barrier()

        def seg_body(s, _):
            off = ebase + s * SEG
            pltpu.sync_copy(src_hbm.at[pl.ds(off, SEG)], srcseg_v)
            pltpu.sync_copy(dst_hbm.at[pl.ds(off, SEG)], dstseg_v)

            @plsc.parallel_loop(0, SEG // 16, unroll=4, carry=jnp.int32(0))
            def kcnt(g, cnt):
                dstv = dstseg_v[pl.ds(g * 16, 16)]
                srcv = srcseg_v[pl.ds(g * 16, 16)]
                local = dstv - lo
                m = local.astype(jnp.uint32) < jnp.uint32(C)
                mi = jnp.where(m, 1, 0).astype(jnp.int32)
                pos = cnt + plsc.cumsum(mi) - 1
                plsc.store_scatter(selsrc_v, [pos >> RBS, pos & (RB - 1)], srcv, mask=m)
                plsc.store_scatter(seldst_v, [pos >> RBS, pos & (RB - 1)], local, mask=m)
                return cnt + jnp.sum(mi)

            nb = (kcnt + (RB - 1)) >> RBS

            g0 = kcnt >> 4
            for t in range(RB // 16 + 1):
                posp = (g0 + t) * 16 + iota
                mp = (posp >= kcnt) & (posp < nb * RB)
                plsc.store_scatter(selsrc_v, [posp >> RBS, posp & (RB - 1)], padsrc_vals, mask=mp)
                plsc.store_scatter(seldst_v, [posp >> RBS, posp & (RB - 1)], paddst_vals, mask=mp)

            def drain_scatter(sem):
                pltpu.make_async_copy(rows0_v, acc_sh.at[seldst_v.at[0]], sem).wait()

            def slot(j, buf, bufn, sem_g, sem_gn, sem_s, sem_sn):
                pltpu.make_async_copy(feat_hbm.at[selsrc_v.at[j]], buf, sem_g).wait()
                pltpu.async_copy(buf, acc_sh.at[seldst_v.at[j]], sem_s, add=True)
                if with_count:
                    pltpu.sync_copy(ones_v, cnt_sh.at[seldst_v.at[j]], add=True)

                @pl.when(j + 1 < nb)
                def _():
                    @pl.when(j >= 1)
                    def _():
                        drain_scatter(sem_sn)

                    pltpu.async_copy(feat_hbm.at[selsrc_v.at[j + 1]], bufn, sem_gn)

            def step(jj, _):
                j0 = jj * 2

                @pl.when(j0 < nb)
                def _():
                    slot(j0, rows0_v, rows1_v, sem0, sem1, sem_s0, sem_s1)

                @pl.when(j0 + 1 < nb)
                def _():
                    slot(j0 + 1, rows1_v, rows0_v, sem1, sem0, sem_s1, sem_s0)

                return 0

            @pl.when(nb > 0)
            def _():
                pltpu.async_copy(feat_hbm.at[selsrc_v.at[0]], rows0_v, sem0)

            lax.fori_loop(0, (nb + 1) >> 1, step, 0)

            @pl.when(nb >= 2)
            def _():
                drain_scatter(sem_s0)
                drain_scatter(sem_s1)

            @pl.when(nb == 1)
            def _():
                drain_scatter(sem_s0)
            return 0

        lax.fori_loop(0, NSEG, seg_body, 0)

        plsc.subcore_barrier()
        r0 = sid * ROWS_PER_TILE
        pltpu.sync_copy(acc_sh.at[pl.ds(r0, ROWS_PER_TILE)],
                        out_hbm.at[pl.ds(lo + r0, ROWS_PER_TILE)])
        if with_count:
            pltpu.sync_copy(cnt_sh.at[pl.ds(r0, ROWS_PER_TILE)],
                            cnt_out_hbm.at[pl.ds(lo + r0, ROWS_PER_TILE)])
        return 0

    lax.fori_loop(0, NCHUNK // NUM_CORES, chunk_body, 0)


def _make_seg_sum(with_count):
    out_type = [jax.ShapeDtypeStruct((OUT_ROWS, H), ADT)]
    if with_count:
        out_type.append(jax.ShapeDtypeStruct((OUT_ROWS, CNTW), ADT))
    mesh = plsc.VectorSubcoreMesh(core_axis_name="c", subcore_axis_name="s")
    scratch = [
        pltpu.VMEM((SEG // RB, RB), jnp.int32),
        pltpu.VMEM((SEG // RB, RB), jnp.int32),
        pltpu.VMEM((SEG,), jnp.int32),
        pltpu.VMEM((SEG,), jnp.int32),
        pltpu.VMEM((RB, H), ADT),
        pltpu.VMEM((RB, H), ADT),
    ]
    if with_count:
        scratch.append(pltpu.VMEM((RB, CNTW), ADT))
    scratch.append(pltpu.VMEM_SHARED((C + TRASH, H), ADT))
    if with_count:
        scratch.append(pltpu.VMEM_SHARED((C + TRASH, CNTW), ADT))
    scratch.extend([pltpu.SemaphoreType.DMA] * 4)
    return pl.kernel(
        lambda *refs: _seg_sum_body(with_count, refs),
        out_type=tuple(out_type),
        mesh=mesh,
        scratch_types=tuple(scratch),
        compiler_params=pltpu.CompilerParams(needs_layout_passes=False,
                                             use_tc_tiling_on_sc=False),
        name="seg_sum_cnt" if with_count else "seg_sum",
    )


_TC_R = 400


def _apply_router_body(s_ref, cnt_ref, h2_ref, featr_ref, wrt_ref, br_ref,
                       outr_ref):
    cnt = jnp.maximum(cnt_ref[...][:, 0:1].astype(jnp.float32), 1.0)
    h1m = s_ref[...].astype(jnp.float32) / cnt
    z = (jnp.dot(h1m, wrt_ref[:H, :], preferred_element_type=jnp.float32,
                 precision=lax.Precision.HIGHEST)
         + jnp.dot(h2_ref[...].astype(jnp.float32), wrt_ref[H:, :], preferred_element_type=jnp.float32,
                   precision=lax.Precision.HIGHEST)
         + br_ref[...])
    outr_ref[...] = featr_ref[...] + jnp.maximum(z, 0.0)


def _apply_packet_body(h1p_ref, featp_ref, wpt_ref, bp_ref, outp_ref):
    p = jnp.dot(h1p_ref[...].astype(jnp.float32), wpt_ref[...], preferred_element_type=jnp.float32,
                precision=lax.Precision.HIGHEST) + bp_ref[...]
    outp_ref[...] = featp_ref[...] + jnp.maximum(p, 0.0)


def _blk(shape):
    return pl.BlockSpec(shape, lambda i: (i, 0))


def _apply_router(s_r, cnt_r, h2_r, feat_router, wrt, br):
    return pl.pallas_call(
        _apply_router_body,
        grid=(N_NODES // _TC_R,),
        in_specs=[_blk((_TC_R, H)), _blk((_TC_R, CNTW)), _blk((_TC_R, H)),
                  _blk((_TC_R, H)),
                  pl.BlockSpec((2 * H, H), lambda i: (0, 0)),
                  pl.BlockSpec((1, H), lambda i: (0, 0))],
        out_specs=_blk((_TC_R, H)),
        out_shape=jax.ShapeDtypeStruct((N_NODES, H), jnp.float32),
    )(s_r, cnt_r, h2_r, feat_router, wrt, br)


def _apply_packet(h1_p, feat_packet, wpt, bp):
    return pl.pallas_call(
        _apply_packet_body,
        grid=(N_NODES // _TC_R,),
        in_specs=[_blk((_TC_R, H)), _blk((_TC_R, H)),
                  pl.BlockSpec((H, H), lambda i: (0, 0)),
                  pl.BlockSpec((1, H), lambda i: (0, 0))],
        out_specs=_blk((_TC_R, H)),
        out_shape=jax.ShapeDtypeStruct((N_NODES, H), jnp.float32),
    )(h1_p, feat_packet, wpt, bp)


def _pad_edges(e):
    padn = E_PAD - E
    src = jnp.concatenate([e[0].astype(jnp.int32),
                           jnp.zeros((padn,), jnp.int32)])
    dst = jnp.concatenate([e[1].astype(jnp.int32),
                           jnp.full((padn,), BIG, jnp.int32)])
    return src, dst


def kernel(feat_router, feat_packet, W_r, b_r, W_p, b_p,
           edge_pass, edge_transfer, edge_connect):
    z800 = jnp.zeros((ROWS_PER_TILE, H), ADT)
    zc800 = jnp.zeros((ROWS_PER_TILE, CNTW), ADT)
    ones = jnp.ones((RB, CNTW), ADT)
    feat_r16 = feat_router.astype(ADT)
    feat_p16 = feat_packet.astype(ADT)

    seg_sum = _make_seg_sum(False)
    seg_sum_cnt = _make_seg_sum(True)

    sp, dp = _pad_edges(edge_pass)
    st, dt = _pad_edges(edge_transfer)
    sc, dc = _pad_edges(edge_connect)

    (h1_packet,) = seg_sum(feat_r16, sp, dp, z800)
    p_new = _apply_packet(h1_packet, feat_packet, W_p.T, b_p.reshape(1, H))
    s_router, cnt16 = seg_sum_cnt(feat_p16, st, dt, z800, zc800, ones)
    (h2_router,) = seg_sum(feat_r16, sc, dc, z800)
    r_new = _apply_router(s_router, cnt16, h2_router, feat_router,
                          W_r.T, b_r.reshape(1, H))
    return r_new, p_new


# speedup vs baseline: 1.1728x; 1.1728x over previous
import functools

import jax
import jax.numpy as jnp
from jax import lax
from jax.experimental import pallas as pl
from jax.experimental.pallas import tpu as pltpu
from jax.experimental.pallas import tpu_sc as plsc

H = 128
N_NODES = 50000
E = 500000

NUM_TILES = 16
NUM_CORES = 2
NCHUNK = 4
C = 12544
OUT_ROWS = NCHUNK * C
TRASH = 8
ROWS_PER_TILE = C // NUM_TILES
SEG = 4096
NSEG = 8
EDGES_PER_TILE = SEG * NSEG
E_PAD = EDGES_PER_TILE * NUM_TILES
RB = 128
RBS = 7
BIG = 1 << 27
CNTW = 32
ADT = jnp.bfloat16


def _seg_sum_body(with_count, refs):
    if with_count:
        (feat_hbm, src_hbm, dst_hbm, z800, zc800, ones_hbm,
         out_hbm, cnt_out_hbm,
         selsrc_v, seldst_v, srcseg_v, dstseg_v, rows0_v, rows1_v, ones_v,
         acc_sh, cnt_sh, sem0, sem1) = refs
    else:
        (feat_hbm, src_hbm, dst_hbm, z800,
         out_hbm,
         selsrc_v, seldst_v, srcseg_v, dstseg_v, rows0_v, rows1_v,
         acc_sh, sem0, sem1) = refs
        cnt_out_hbm = zc800 = ones_hbm = ones_v = cnt_sh = None

    core = lax.axis_index("c")
    sid = lax.axis_index("s")
    iota = lax.iota(jnp.int32, 16)
    padsrc_vals = iota * 97 + sid * 16
    paddst_vals = C + (iota & 7)

    if with_count:
        pltpu.sync_copy(ones_hbm, ones_v)
    ebase = sid * EDGES_PER_TILE

    def chunk_body(k, _):
        lo = (core * (NCHUNK // NUM_CORES) + k) * C
        plsc.subcore_barrier()
        pltpu.sync_copy(z800, acc_sh.at[pl.ds(sid * ROWS_PER_TILE, ROWS_PER_TILE)])
        if with_count:
            pltpu.sync_copy(zc800, cnt_sh.at[pl.ds(sid * ROWS_PER_TILE, ROWS_PER_TILE)])
        plsc.subcore_barrier()

        def seg_body(s, _):
            off = ebase + s * SEG
            pltpu.sync_copy(src_hbm.at[pl.ds(off, SEG)], srcseg_v)
            pltpu.sync_copy(dst_hbm.at[pl.ds(off, SEG)], dstseg_v)

            @plsc.parallel_loop(0, SEG // 16, unroll=4, carry=jnp.int32(0))
            def kcnt(g, cnt):
                dstv = dstseg_v[pl.ds(g * 16, 16)]
                srcv = srcseg_v[pl.ds(g * 16, 16)]
                local = dstv - lo
                m = local.astype(jnp.uint32) < jnp.uint32(C)
                mi = jnp.where(m, 1, 0).astype(jnp.int32)
                pos = cnt + plsc.cumsum(mi) - 1
                plsc.store_scatter(selsrc_v, [pos >> RBS, pos & (RB - 1)], srcv, mask=m)
                plsc.store_scatter(seldst_v, [pos >> RBS, pos & (RB - 1)], local, mask=m)
                return cnt + jnp.sum(mi)

            nb = (kcnt + (RB - 1)) >> RBS

            g0 = kcnt >> 4
            for t in range(RB // 16 + 1):
                posp = (g0 + t) * 16 + iota
                mp = (posp >= kcnt) & (posp < nb * RB)
                plsc.store_scatter(selsrc_v, [posp >> RBS, posp & (RB - 1)], padsrc_vals, mask=mp)
                plsc.store_scatter(seldst_v, [posp >> RBS, posp & (RB - 1)], paddst_vals, mask=mp)

            def slot(j, nxt, buf, bufn, sem, semn):
                pltpu.make_async_copy(feat_hbm.at[selsrc_v.at[j]], buf, sem).wait()

                @pl.when(nxt < nb)
                def _():
                    pltpu.async_copy(feat_hbm.at[selsrc_v.at[nxt]], bufn, semn)

                pltpu.sync_copy(buf, acc_sh.at[seldst_v.at[j]], add=True)
                if with_count:
                    pltpu.sync_copy(ones_v, cnt_sh.at[seldst_v.at[j]], add=True)

            def step(jj, _):
                j0 = jj * 2

                @pl.when(j0 < nb)
                def _():
                    slot(j0, j0 + 1, rows0_v, rows1_v, sem0, sem1)

                @pl.when(j0 + 1 < nb)
                def _():
                    slot(j0 + 1, j0 + 2, rows1_v, rows0_v, sem1, sem0)

                return 0

            @pl.when(nb > 0)
            def _():
                pltpu.async_copy(feat_hbm.at[selsrc_v.at[0]], rows0_v, sem0)

            lax.fori_loop(0, (nb + 1) >> 1, step, 0)
            return 0

        lax.fori_loop(0, NSEG, seg_body, 0)

        plsc.subcore_barrier()
        r0 = sid * ROWS_PER_TILE
        pltpu.sync_copy(acc_sh.at[pl.ds(r0, ROWS_PER_TILE)],
                        out_hbm.at[pl.ds(lo + r0, ROWS_PER_TILE)])
        if with_count:
            pltpu.sync_copy(cnt_sh.at[pl.ds(r0, ROWS_PER_TILE)],
                            cnt_out_hbm.at[pl.ds(lo + r0, ROWS_PER_TILE)])
        return 0

    lax.fori_loop(0, NCHUNK // NUM_CORES, chunk_body, 0)


def _make_seg_sum(with_count):
    out_type = [jax.ShapeDtypeStruct((OUT_ROWS, H), ADT)]
    if with_count:
        out_type.append(jax.ShapeDtypeStruct((OUT_ROWS, CNTW), ADT))
    mesh = plsc.VectorSubcoreMesh(core_axis_name="c", subcore_axis_name="s")
    scratch = [
        pltpu.VMEM((SEG // RB, RB), jnp.int32),
        pltpu.VMEM((SEG // RB, RB), jnp.int32),
        pltpu.VMEM((SEG,), jnp.int32),
        pltpu.VMEM((SEG,), jnp.int32),
        pltpu.VMEM((RB, H), ADT),
        pltpu.VMEM((RB, H), ADT),
    ]
    if with_count:
        scratch.append(pltpu.VMEM((RB, CNTW), ADT))
    scratch.append(pltpu.VMEM_SHARED((C + TRASH, H), ADT))
    if with_count:
        scratch.append(pltpu.VMEM_SHARED((C + TRASH, CNTW), ADT))
    scratch.extend([pltpu.SemaphoreType.DMA] * 2)
    return pl.kernel(
        lambda *refs: _seg_sum_body(with_count, refs),
        out_type=tuple(out_type),
        mesh=mesh,
        scratch_types=tuple(scratch),
        compiler_params=pltpu.CompilerParams(needs_layout_passes=False,
                                             use_tc_tiling_on_sc=False),
        name="seg_sum_cnt" if with_count else "seg_sum",
    )


_TC_R = 2000


def _apply_router_body(s_ref, cnt_ref, h2_ref, featr_ref, wrt_ref, br_ref,
                       outr_ref):
    cnt = jnp.maximum(cnt_ref[...][:, 0:1].astype(jnp.float32), 1.0)
    h1m = s_ref[...].astype(jnp.float32) / cnt
    z = (jnp.dot(h1m, wrt_ref[:H, :], preferred_element_type=jnp.float32,
                 precision=lax.Precision.HIGHEST)
         + jnp.dot(h2_ref[...].astype(jnp.float32), wrt_ref[H:, :], preferred_element_type=jnp.float32,
                   precision=lax.Precision.HIGHEST)
         + br_ref[...])
    outr_ref[...] = featr_ref[...] + jnp.maximum(z, 0.0)


def _apply_packet_body(h1p_ref, featp_ref, wpt_ref, bp_ref, outp_ref):
    p = jnp.dot(h1p_ref[...].astype(jnp.float32), wpt_ref[...], preferred_element_type=jnp.float32,
                precision=lax.Precision.HIGHEST) + bp_ref[...]
    outp_ref[...] = featp_ref[...] + jnp.maximum(p, 0.0)


def _blk(shape):
    return pl.BlockSpec(shape, lambda i: (i, 0))


def _apply_router(s_r, cnt_r, h2_r, feat_router, wrt, br):
    return pl.pallas_call(
        _apply_router_body,
        grid=(N_NODES // _TC_R,),
        in_specs=[_blk((_TC_R, H)), _blk((_TC_R, CNTW)), _blk((_TC_R, H)),
                  _blk((_TC_R, H)),
                  pl.BlockSpec((2 * H, H), lambda i: (0, 0)),
                  pl.BlockSpec((1, H), lambda i: (0, 0))],
        out_specs=_blk((_TC_R, H)),
        out_shape=jax.ShapeDtypeStruct((N_NODES, H), jnp.float32),
    )(s_r, cnt_r, h2_r, feat_router, wrt, br)


def _apply_packet(h1_p, feat_packet, wpt, bp):
    return pl.pallas_call(
        _apply_packet_body,
        grid=(N_NODES // _TC_R,),
        in_specs=[_blk((_TC_R, H)), _blk((_TC_R, H)),
                  pl.BlockSpec((H, H), lambda i: (0, 0)),
                  pl.BlockSpec((1, H), lambda i: (0, 0))],
        out_specs=_blk((_TC_R, H)),
        out_shape=jax.ShapeDtypeStruct((N_NODES, H), jnp.float32),
    )(h1_p, feat_packet, wpt, bp)


def _pad_edges(e):
    padn = E_PAD - E
    src = jnp.concatenate([e[0].astype(jnp.int32),
                           jnp.zeros((padn,), jnp.int32)])
    dst = jnp.concatenate([e[1].astype(jnp.int32),
                           jnp.full((padn,), BIG, jnp.int32)])
    return src, dst


def kernel(feat_router, feat_packet, W_r, b_r, W_p, b_p,
           edge_pass, edge_transfer, edge_connect):
    z800 = jnp.zeros((ROWS_PER_TILE, H), ADT)
    zc800 = jnp.zeros((ROWS_PER_TILE, CNTW), ADT)
    ones = jnp.ones((RB, CNTW), ADT)
    feat_r16 = feat_router.astype(ADT)
    feat_p16 = feat_packet.astype(ADT)

    seg_sum = _make_seg_sum(False)
    seg_sum_cnt = _make_seg_sum(True)

    sp, dp = _pad_edges(edge_pass)
    st, dt = _pad_edges(edge_transfer)
    sc, dc = _pad_edges(edge_connect)

    (h1_packet,) = seg_sum(feat_r16, sp, dp, z800)
    p_new = _apply_packet(h1_packet, feat_packet, W_p.T, b_p.reshape(1, H))
    st, dt, _ = lax.optimization_barrier((st, dt, h1_packet))
    sc, dc, _ = lax.optimization_barrier((sc, dc, h1_packet))
    s_router, cnt16 = seg_sum_cnt(feat_p16, st, dt, z800, zc800, ones)
    (h2_router,) = seg_sum(feat_r16, sc, dc, z800)
    r_new = _apply_router(s_router, cnt16, h2_router, feat_router,
                          W_r.T, b_r.reshape(1, H))
    return r_new, p_new


# speedup vs baseline: 1.1803x; 1.0064x over previous
import functools

import jax
import jax.numpy as jnp
from jax import lax
from jax.experimental import pallas as pl
from jax.experimental.pallas import tpu as pltpu
from jax.experimental.pallas import tpu_sc as plsc

H = 128
N_NODES = 50000
E = 500000

NUM_TILES = 16
NUM_CORES = 2
NCHUNK = 4
C = 12544
OUT_ROWS = NCHUNK * C
TRASH = 8
ROWS_PER_TILE = C // NUM_TILES
SEG = 4096
NSEG = 8
EDGES_PER_TILE = SEG * NSEG
E_PAD = EDGES_PER_TILE * NUM_TILES
RB = 128
RBS = 7
BIG = 1 << 27
CNTW = 32
ADT = jnp.bfloat16


def _seg_sum_body(with_count, refs):
    if with_count:
        (feat_hbm, src_hbm, dst_hbm, z800, zc800, ones_hbm,
         out_hbm, cnt_out_hbm,
         selsrc_v, seldst_v, srcseg_v, dstseg_v, rows0_v, rows1_v, ones_v,
         acc_sh, cnt_sh, sem0, sem1) = refs
    else:
        (feat_hbm, src_hbm, dst_hbm, z800,
         out_hbm,
         selsrc_v, seldst_v, srcseg_v, dstseg_v, rows0_v, rows1_v,
         acc_sh, sem0, sem1) = refs
        cnt_out_hbm = zc800 = ones_hbm = ones_v = cnt_sh = None

    core = lax.axis_index("c")
    sid = lax.axis_index("s")
    iota = lax.iota(jnp.int32, 16)
    padsrc_vals = iota * 97 + sid * 16
    paddst_vals = C + (iota & 7)

    if with_count:
        pltpu.sync_copy(ones_hbm, ones_v)
    ebase = sid * EDGES_PER_TILE

    def chunk_body(k, _):
        lo = (core * (NCHUNK // NUM_CORES) + k) * C
        plsc.subcore_barrier()
        pltpu.sync_copy(z800, acc_sh.at[pl.ds(sid * ROWS_PER_TILE, ROWS_PER_TILE)])
        if with_count:
            pltpu.sync_copy(zc800, cnt_sh.at[pl.ds(sid * ROWS_PER_TILE, ROWS_PER_TILE)])
        plsc.subcore_barrier()

        def seg_body(s, _):
            off = ebase + s * SEG
            pltpu.sync_copy(src_hbm.at[pl.ds(off, SEG)], srcseg_v)
            pltpu.sync_copy(dst_hbm.at[pl.ds(off, SEG)], dstseg_v)

            @plsc.parallel_loop(0, SEG // 16, unroll=4, carry=jnp.int32(0))
            def kcnt(g, cnt):
                dstv = dstseg_v[pl.ds(g * 16, 16)]
                srcv = srcseg_v[pl.ds(g * 16, 16)]
                local = dstv - lo
                m = local.astype(jnp.uint32) < jnp.uint32(C)
                mi = jnp.where(m, 1, 0).astype(jnp.int32)
                pos = cnt + plsc.cumsum(mi) - 1
                plsc.store_scatter(selsrc_v, [pos >> RBS, pos & (RB - 1)], srcv, mask=m)
                plsc.store_scatter(seldst_v, [pos >> RBS, pos & (RB - 1)], local, mask=m)
                return cnt + jnp.sum(mi)

            nb = (kcnt + (RB - 1)) >> RBS

            g0 = kcnt >> 4
            for t in range(RB // 16 + 1):
                posp = (g0 + t) * 16 + iota
                mp = (posp >= kcnt) & (posp < nb * RB)
                plsc.store_scatter(selsrc_v, [posp >> RBS, posp & (RB - 1)], padsrc_vals, mask=mp)
                plsc.store_scatter(seldst_v, [posp >> RBS, posp & (RB - 1)], paddst_vals, mask=mp)

            def slot(j, nxt, buf, bufn, sem, semn):
                pltpu.make_async_copy(feat_hbm.at[selsrc_v.at[j]], buf, sem).wait()

                @pl.when(nxt < nb)
                def _():
                    pltpu.async_copy(feat_hbm.at[selsrc_v.at[nxt]], bufn, semn)

                pltpu.sync_copy(buf, acc_sh.at[seldst_v.at[j]], add=True)
                if with_count:
                    pltpu.sync_copy(ones_v, cnt_sh.at[seldst_v.at[j]], add=True)

            def step(jj, _):
                j0 = jj * 2

                @pl.when(j0 < nb)
                def _():
                    slot(j0, j0 + 1, rows0_v, rows1_v, sem0, sem1)

                @pl.when(j0 + 1 < nb)
                def _():
                    slot(j0 + 1, j0 + 2, rows1_v, rows0_v, sem1, sem0)

                return 0

            @pl.when(nb > 0)
            def _():
                pltpu.async_copy(feat_hbm.at[selsrc_v.at[0]], rows0_v, sem0)

            lax.fori_loop(0, (nb + 1) >> 1, step, 0)
            return 0

        lax.fori_loop(0, NSEG, seg_body, 0)

        plsc.subcore_barrier()
        r0 = sid * ROWS_PER_TILE
        pltpu.sync_copy(acc_sh.at[pl.ds(r0, ROWS_PER_TILE)],
                        out_hbm.at[pl.ds(lo + r0, ROWS_PER_TILE)])
        if with_count:
            pltpu.sync_copy(cnt_sh.at[pl.ds(r0, ROWS_PER_TILE)],
                            cnt_out_hbm.at[pl.ds(lo + r0, ROWS_PER_TILE)])
        return 0

    lax.fori_loop(0, NCHUNK // NUM_CORES, chunk_body, 0)


def _make_seg_sum(with_count):
    out_type = [jax.ShapeDtypeStruct((OUT_ROWS, H), ADT)]
    if with_count:
        out_type.append(jax.ShapeDtypeStruct((OUT_ROWS, CNTW), ADT))
    mesh = plsc.VectorSubcoreMesh(core_axis_name="c", subcore_axis_name="s")
    scratch = [
        pltpu.VMEM((SEG // RB, RB), jnp.int32),
        pltpu.VMEM((SEG // RB, RB), jnp.int32),
        pltpu.VMEM((SEG,), jnp.int32),
        pltpu.VMEM((SEG,), jnp.int32),
        pltpu.VMEM((RB, H), ADT),
        pltpu.VMEM((RB, H), ADT),
    ]
    if with_count:
        scratch.append(pltpu.VMEM((RB, CNTW), ADT))
    scratch.append(pltpu.VMEM_SHARED((C + TRASH, H), ADT))
    if with_count:
        scratch.append(pltpu.VMEM_SHARED((C + TRASH, CNTW), ADT))
    scratch.extend([pltpu.SemaphoreType.DMA] * 2)
    return pl.kernel(
        lambda *refs: _seg_sum_body(with_count, refs),
        out_type=tuple(out_type),
        mesh=mesh,
        scratch_types=tuple(scratch),
        compiler_params=pltpu.CompilerParams(needs_layout_passes=False,
                                             use_tc_tiling_on_sc=False),
        name="seg_sum_cnt" if with_count else "seg_sum",
    )


_TC_R = 2000


def _apply_router_body(s_ref, cnt_ref, h2_ref, featr_ref, wrt_ref, br_ref,
                       outr_ref):
    cnt = jnp.maximum(cnt_ref[...][:, 0:1].astype(jnp.float32), 1.0)
    h1m = s_ref[...].astype(jnp.float32) / cnt
    z = (jnp.dot(h1m, wrt_ref[:H, :], preferred_element_type=jnp.float32,
                 precision=lax.Precision.HIGHEST)
         + jnp.dot(h2_ref[...].astype(jnp.float32), wrt_ref[H:, :], preferred_element_type=jnp.float32,
                   precision=lax.Precision.HIGHEST)
         + br_ref[...])
    outr_ref[...] = featr_ref[...] + jnp.maximum(z, 0.0)


def _apply_packet_body(h1p_ref, featp_ref, wpt_ref, bp_ref, outp_ref):
    p = jnp.dot(h1p_ref[...].astype(jnp.float32), wpt_ref[...], preferred_element_type=jnp.float32,
                precision=lax.Precision.HIGHEST) + bp_ref[...]
    outp_ref[...] = featp_ref[...] + jnp.maximum(p, 0.0)


def _blk(shape):
    return pl.BlockSpec(shape, lambda i: (i, 0))


def _apply_router(s_r, cnt_r, h2_r, feat_router, wrt, br):
    return pl.pallas_call(
        _apply_router_body,
        grid=(N_NODES // _TC_R,),
        in_specs=[_blk((_TC_R, H)), _blk((_TC_R, CNTW)), _blk((_TC_R, H)),
                  _blk((_TC_R, H)),
                  pl.BlockSpec((2 * H, H), lambda i: (0, 0)),
                  pl.BlockSpec((1, H), lambda i: (0, 0))],
        out_specs=_blk((_TC_R, H)),
        out_shape=jax.ShapeDtypeStruct((N_NODES, H), jnp.float32),
    )(s_r, cnt_r, h2_r, feat_router, wrt, br)


def _apply_packet(h1_p, feat_packet, wpt, bp):
    return pl.pallas_call(
        _apply_packet_body,
        grid=(N_NODES // _TC_R,),
        in_specs=[_blk((_TC_R, H)), _blk((_TC_R, H)),
                  pl.BlockSpec((H, H), lambda i: (0, 0)),
                  pl.BlockSpec((1, H), lambda i: (0, 0))],
        out_specs=_blk((_TC_R, H)),
        out_shape=jax.ShapeDtypeStruct((N_NODES, H), jnp.float32),
    )(h1_p, feat_packet, wpt, bp)


def _pad_edges(e):
    padn = E_PAD - E
    src = jnp.concatenate([e[0].astype(jnp.int32),
                           jnp.zeros((padn,), jnp.int32)])
    dst = jnp.concatenate([e[1].astype(jnp.int32),
                           jnp.full((padn,), BIG, jnp.int32)])
    return src, dst


def kernel(feat_router, feat_packet, W_r, b_r, W_p, b_p,
           edge_pass, edge_transfer, edge_connect):
    z800 = jnp.zeros((ROWS_PER_TILE, H), ADT)
    zc800 = jnp.zeros((ROWS_PER_TILE, CNTW), ADT)
    ones = jnp.ones((RB, CNTW), ADT)
    feat_r16 = feat_router.astype(ADT)
    feat_p16 = feat_packet.astype(ADT)

    seg_sum = _make_seg_sum(False)
    seg_sum_cnt = _make_seg_sum(True)

    sp, dp = _pad_edges(edge_pass)
    st, dt = _pad_edges(edge_transfer)
    sc, dc = _pad_edges(edge_connect)

    (h1_packet,) = seg_sum(feat_r16, sp, dp, z800)
    p_new = _apply_packet(h1_packet, feat_packet, W_p.T, b_p.reshape(1, H))
    sc, dc, _ = lax.optimization_barrier((sc, dc, h1_packet))
    s_router, cnt16 = seg_sum_cnt(feat_p16, st, dt, z800, zc800, ones)
    (h2_router,) = seg_sum(feat_r16, sc, dc, z800)
    r_new = _apply_router(s_router, cnt16, h2_router, feat_router,
                          W_r.T, b_r.reshape(1, H))
    return r_new, p_new


# speedup vs baseline: 1.2638x; 1.0708x over previous
import functools

import jax
import jax.numpy as jnp
from jax import lax
from jax.experimental import pallas as pl
from jax.experimental.pallas import tpu as pltpu
from jax.experimental.pallas import tpu_sc as plsc

H = 128
N_NODES = 50000
E = 500000

NUM_TILES = 16
NUM_CORES = 2
NCHUNK = 4
C = 12544
OUT_ROWS = NCHUNK * C
TRASH = 8
ROWS_PER_TILE = C // NUM_TILES
SEG = 8192
NSEG = 4
EDGES_PER_TILE = SEG * NSEG
E_PAD = EDGES_PER_TILE * NUM_TILES
RB = 128
RBS = 7
BIG = 1 << 27
CNTW = 32
ADT = jnp.bfloat16


def _seg_sum_body(with_count, refs):
    if with_count:
        (feat_hbm, src_hbm, dst_hbm, z800, zc800, ones_hbm,
         out_hbm, cnt_out_hbm,
         selsrc_v, seldst_v, srcseg_v, dstseg_v, rows0_v, rows1_v, ones_v,
         acc_sh, cnt_sh, sem0, sem1) = refs
    else:
        (feat_hbm, src_hbm, dst_hbm, z800,
         out_hbm,
         selsrc_v, seldst_v, srcseg_v, dstseg_v, rows0_v, rows1_v,
         acc_sh, sem0, sem1) = refs
        cnt_out_hbm = zc800 = ones_hbm = ones_v = cnt_sh = None

    core = lax.axis_index("c")
    sid = lax.axis_index("s")
    iota = lax.iota(jnp.int32, 16)
    padsrc_vals = iota * 97 + sid * 16
    paddst_vals = C + (iota & 7)

    if with_count:
        pltpu.sync_copy(ones_hbm, ones_v)
    ebase = sid * EDGES_PER_TILE

    def chunk_body(k, _):
        lo = (core * (NCHUNK // NUM_CORES) + k) * C
        plsc.subcore_barrier()
        pltpu.sync_copy(z800, acc_sh.at[pl.ds(sid * ROWS_PER_TILE, ROWS_PER_TILE)])
        if with_count:
            pltpu.sync_copy(zc800, cnt_sh.at[pl.ds(sid * ROWS_PER_TILE, ROWS_PER_TILE)])
        plsc.subcore_barrier()

        def seg_body(s, _):
            off = ebase + s * SEG
            pltpu.sync_copy(src_hbm.at[pl.ds(off, SEG)], srcseg_v)
            pltpu.sync_copy(dst_hbm.at[pl.ds(off, SEG)], dstseg_v)

            @plsc.parallel_loop(0, SEG // 16, unroll=8, carry=jnp.int32(0))
            def kcnt(g, cnt):
                dstv = dstseg_v[pl.ds(g * 16, 16)]
                srcv = srcseg_v[pl.ds(g * 16, 16)]
                local = dstv - lo
                m = local.astype(jnp.uint32) < jnp.uint32(C)
                mi = jnp.where(m, 1, 0).astype(jnp.int32)
                pos = cnt + plsc.cumsum(mi) - 1
                plsc.store_scatter(selsrc_v, [pos >> RBS, pos & (RB - 1)], srcv, mask=m)
                plsc.store_scatter(seldst_v, [pos >> RBS, pos & (RB - 1)], local, mask=m)
                return cnt + jnp.sum(mi)

            nb = (kcnt + (RB - 1)) >> RBS

            g0 = kcnt >> 4
            for t in range(RB // 16 + 1):
                posp = (g0 + t) * 16 + iota
                mp = (posp >= kcnt) & (posp < nb * RB)
                plsc.store_scatter(selsrc_v, [posp >> RBS, posp & (RB - 1)], padsrc_vals, mask=mp)
                plsc.store_scatter(seldst_v, [posp >> RBS, posp & (RB - 1)], paddst_vals, mask=mp)

            def slot(j, nxt, buf, bufn, sem, semn):
                pltpu.make_async_copy(feat_hbm.at[selsrc_v.at[j]], buf, sem).wait()

                @pl.when(nxt < nb)
                def _():
                    pltpu.async_copy(feat_hbm.at[selsrc_v.at[nxt]], bufn, semn)

                pltpu.sync_copy(buf, acc_sh.at[seldst_v.at[j]], add=True)
                if with_count:
                    pltpu.sync_copy(ones_v, cnt_sh.at[seldst_v.at[j]], add=True)

            def step(jj, _):
                j0 = jj * 2

                @pl.when(j0 < nb)
                def _():
                    slot(j0, j0 + 1, rows0_v, rows1_v, sem0, sem1)

                @pl.when(j0 + 1 < nb)
                def _():
                    slot(j0 + 1, j0 + 2, rows1_v, rows0_v, sem1, sem0)

                return 0

            @pl.when(nb > 0)
            def _():
                pltpu.async_copy(feat_hbm.at[selsrc_v.at[0]], rows0_v, sem0)

            lax.fori_loop(0, (nb + 1) >> 1, step, 0)
            return 0

        lax.fori_loop(0, NSEG, seg_body, 0)

        plsc.subcore_barrier()
        r0 = sid * ROWS_PER_TILE
        pltpu.sync_copy(acc_sh.at[pl.ds(r0, ROWS_PER_TILE)],
                        out_hbm.at[pl.ds(lo + r0, ROWS_PER_TILE)])
        if with_count:
            pltpu.sync_copy(cnt_sh.at[pl.ds(r0, ROWS_PER_TILE)],
                            cnt_out_hbm.at[pl.ds(lo + r0, ROWS_PER_TILE)])
        return 0

    lax.fori_loop(0, NCHUNK // NUM_CORES, chunk_body, 0)


def _make_seg_sum(with_count):
    out_type = [jax.ShapeDtypeStruct((OUT_ROWS, H), ADT)]
    if with_count:
        out_type.append(jax.ShapeDtypeStruct((OUT_ROWS, CNTW), ADT))
    mesh = plsc.VectorSubcoreMesh(core_axis_name="c", subcore_axis_name="s")
    scratch = [
        pltpu.VMEM((SEG // RB, RB), jnp.int32),
        pltpu.VMEM((SEG // RB, RB), jnp.int32),
        pltpu.VMEM((SEG,), jnp.int32),
        pltpu.VMEM((SEG,), jnp.int32),
        pltpu.VMEM((RB, H), ADT),
        pltpu.VMEM((RB, H), ADT),
    ]
    if with_count:
        scratch.append(pltpu.VMEM((RB, CNTW), ADT))
    scratch.append(pltpu.VMEM_SHARED((C + TRASH, H), ADT))
    if with_count:
        scratch.append(pltpu.VMEM_SHARED((C + TRASH, CNTW), ADT))
    scratch.extend([pltpu.SemaphoreType.DMA] * 2)
    return pl.kernel(
        lambda *refs: _seg_sum_body(with_count, refs),
        out_type=tuple(out_type),
        mesh=mesh,
        scratch_types=tuple(scratch),
        compiler_params=pltpu.CompilerParams(needs_layout_passes=False,
                                             use_tc_tiling_on_sc=False),
        name="seg_sum_cnt" if with_count else "seg_sum",
    )


_TC_R = 2000


def _apply_router_body(s_ref, cnt_ref, h2_ref, featr_ref, wrt_ref, br_ref,
                       outr_ref):
    cnt = jnp.maximum(cnt_ref[...][:, 0:1].astype(jnp.float32), 1.0)
    h1m = s_ref[...].astype(jnp.float32) / cnt
    z = (jnp.dot(h1m, wrt_ref[:H, :], preferred_element_type=jnp.float32,
                 precision=lax.Precision.HIGHEST)
         + jnp.dot(h2_ref[...].astype(jnp.float32), wrt_ref[H:, :], preferred_element_type=jnp.float32,
                   precision=lax.Precision.HIGHEST)
         + br_ref[...])
    outr_ref[...] = featr_ref[...] + jnp.maximum(z, 0.0)


def _apply_packet_body(h1p_ref, featp_ref, wpt_ref, bp_ref, outp_ref):
    p = jnp.dot(h1p_ref[...].astype(jnp.float32), wpt_ref[...], preferred_element_type=jnp.float32,
                precision=lax.Precision.HIGHEST) + bp_ref[...]
    outp_ref[...] = featp_ref[...] + jnp.maximum(p, 0.0)


def _blk(shape):
    return pl.BlockSpec(shape, lambda i: (i, 0))


def _apply_router(s_r, cnt_r, h2_r, feat_router, wrt, br):
    return pl.pallas_call(
        _apply_router_body,
        grid=(N_NODES // _TC_R,),
        in_specs=[_blk((_TC_R, H)), _blk((_TC_R, CNTW)), _blk((_TC_R, H)),
                  _blk((_TC_R, H)),
                  pl.BlockSpec((2 * H, H), lambda i: (0, 0)),
                  pl.BlockSpec((1, H), lambda i: (0, 0))],
        out_specs=_blk((_TC_R, H)),
        out_shape=jax.ShapeDtypeStruct((N_NODES, H), jnp.float32),
    )(s_r, cnt_r, h2_r, feat_router, wrt, br)


def _apply_packet(h1_p, feat_packet, wpt, bp):
    return pl.pallas_call(
        _apply_packet_body,
        grid=(N_NODES // _TC_R,),
        in_specs=[_blk((_TC_R, H)), _blk((_TC_R, H)),
                  pl.BlockSpec((H, H), lambda i: (0, 0)),
                  pl.BlockSpec((1, H), lambda i: (0, 0))],
        out_specs=_blk((_TC_R, H)),
        out_shape=jax.ShapeDtypeStruct((N_NODES, H), jnp.float32),
    )(h1_p, feat_packet, wpt, bp)


def _pad_edges(e):
    padn = E_PAD - E
    src = jnp.concatenate([e[0].astype(jnp.int32),
                           jnp.zeros((padn,), jnp.int32)])
    dst = jnp.concatenate([e[1].astype(jnp.int32),
                           jnp.full((padn,), BIG, jnp.int32)])
    return src, dst


def kernel(feat_router, feat_packet, W_r, b_r, W_p, b_p,
           edge_pass, edge_transfer, edge_connect):
    z800 = jnp.zeros((ROWS_PER_TILE, H), ADT)
    zc800 = jnp.zeros((ROWS_PER_TILE, CNTW), ADT)
    ones = jnp.ones((RB, CNTW), ADT)
    feat_r16 = feat_router.astype(ADT)
    feat_p16 = feat_packet.astype(ADT)

    seg_sum = _make_seg_sum(False)
    seg_sum_cnt = _make_seg_sum(True)

    sp, dp = _pad_edges(edge_pass)
    st, dt = _pad_edges(edge_transfer)
    sc, dc = _pad_edges(edge_connect)

    (h1_packet,) = seg_sum(feat_r16, sp, dp, z800)
    p_new = _apply_packet(h1_packet, feat_packet, W_p.T, b_p.reshape(1, H))
    sc, dc, _ = lax.optimization_barrier((sc, dc, h1_packet))
    s_router, cnt16 = seg_sum_cnt(feat_p16, st, dt, z800, zc800, ones)
    (h2_router,) = seg_sum(feat_r16, sc, dc, z800)
    r_new = _apply_router(s_router, cnt16, h2_router, feat_router,
                          W_r.T, b_r.reshape(1, H))
    return r_new, p_new
